# baseline (speedup 1.0000x reference)
import functools

import jax
import jax.numpy as jnp
from jax import lax
from jax.experimental import pallas as pl
from jax.experimental.pallas import tpu as pltpu
from jax.experimental.pallas import tpu_sc as plsc

VOCAB = 100000
D = 128
B = 1024
S = 200
N = B * S
EPS = 1e-6

NUM_CORES = 2
NUM_SUBCORES = 16
NW = NUM_CORES * NUM_SUBCORES
ROWS_PER_W = N // NW
CHUNK = 128
NCHUNK = ROWS_PER_W // CHUNK


def _gather(src_flat, W):
    mesh = plsc.VectorSubcoreMesh(core_axis_name="c", subcore_axis_name="s")

    @functools.partial(
        pl.kernel,
        mesh=mesh,
        out_type=jax.ShapeDtypeStruct((N, D), jnp.float32),
        scratch_types=[
            pltpu.VMEM((ROWS_PER_W,), jnp.int32),
            pltpu.VMEM((CHUNK, D), jnp.float32),
            pltpu.VMEM((CHUNK, D), jnp.float32),
            pltpu.SemaphoreType.DMA,
            pltpu.SemaphoreType.DMA,
            pltpu.SemaphoreType.DMA,
            pltpu.SemaphoreType.DMA,
        ],
    )
    def body(w_hbm, i_hbm, o_hbm, idx_v, b0, b1, gs0, gs1, os0, os1):
        wid = lax.axis_index("s") * NUM_CORES + lax.axis_index("c")
        base = wid * ROWS_PER_W
        pltpu.sync_copy(i_hbm.at[pl.ds(base, ROWS_PER_W)], idx_v)

        bufs = (b0, b1)
        gsems = (gs0, gs1)
        osems = (os0, os1)

        def startg(buf, gsem, i):
            pltpu.async_copy(w_hbm.at[idx_v.at[pl.ds(i * CHUNK, CHUNK)]],
                             buf, gsem)

        def waitg(buf, gsem):
            pltpu.make_async_copy(w_hbm.at[pl.ds(0, CHUNK)], buf, gsem).wait()

        def starto(buf, osem, i):
            pltpu.async_copy(buf, o_hbm.at[pl.ds(base + i * CHUNK, CHUNK)],
                             osem)

        def waito(buf, osem):
            pltpu.make_async_copy(buf, o_hbm.at[pl.ds(0, CHUNK)], osem).wait()

        startg(bufs[0], gsems[0], 0)
        startg(bufs[1], gsems[1], 1)

        def step(k, _):
            for b in range(2):
                i = 2 * k + b
                waitg(bufs[b], gsems[b])
                starto(bufs[b], osems[b], i)
                waito(bufs[b], osems[b])

                @pl.when(i + 2 < NCHUNK)
                def _():
                    startg(bufs[b], gsems[b], i + 2)
            return 0

        lax.fori_loop(0, NCHUNK // 2, step, 0)

    return body(W, src_flat)


BB = 16


def _ln(x, g, bta):
    s = jnp.sum(x, axis=-1)
    q = jnp.sum(x * x, axis=-1)
    mean = s * (1.0 / D)
    var = (q - s * mean) * (1.0 / (D - 1))
    r = lax.rsqrt(jnp.maximum(var, 1e-30))
    inv = r - EPS * (r * r)
    return (x - mean[..., None]) * (inv[..., None] * g) + bta


def _ln0_kernel(raw_ref, g_ref, b_ref, o_ref):
    o_ref[...] = _ln(raw_ref[...], g_ref[...], b_ref[...])


def _ln1_kernel(raw_ref, seg_ref, pos_ref, segtab_ref, g_ref, b_ref, o_ref):
    seg = seg_ref[...][..., None]
    st = segtab_ref[...]
    segemb = jnp.where(seg == 0, st[0], jnp.where(seg == 1, st[1], st[2]))
    x = raw_ref[...] + pos_ref[...][None, :, :] + segemb
    o_ref[...] = _ln(x, g_ref[...], b_ref[...])


_BLK = pl.BlockSpec((BB, S, D), lambda i: (i, 0, 0))
_VEC = pl.BlockSpec((1, D), lambda i: (0, 0))


def _ln0_call(raw, gamma, beta):
    return pl.pallas_call(
        _ln0_kernel,
        grid=(B // BB,),
        in_specs=[_BLK, _VEC, _VEC],
        out_specs=_BLK,
        out_shape=jax.ShapeDtypeStruct((B, S, D), jnp.float32),
    )(raw, gamma, beta)


def _ln1_call(raw, seg_1, pos_slice, seg_table, gamma, beta):
    return pl.pallas_call(
        _ln1_kernel,
        grid=(B // BB,),
        in_specs=[
            _BLK,
            pl.BlockSpec((BB, S), lambda i: (i, 0)),
            pl.BlockSpec((S, D), lambda i: (0, 0)),
            pl.BlockSpec((3, D), lambda i: (0, 0)),
            _VEC,
            _VEC,
        ],
        out_specs=_BLK,
        out_shape=jax.ShapeDtypeStruct((B, S, D), jnp.float32),
    )(raw, seg_1, pos_slice, seg_table, gamma, beta)


def kernel(src_0, src_1, seg_0, seg_1, W0, gamma0, beta0, W1, pos_table,
           seg_table, gamma1, beta1):
    src0_flat = src_0.reshape(N).astype(jnp.int32)
    src1_flat = src_1.reshape(N).astype(jnp.int32)
    raw0 = _gather(src0_flat, W0).reshape(B, S, D)
    raw1 = _gather(src1_flat, W1).reshape(B, S, D)
    e0 = _ln0_call(raw0, gamma0.reshape(1, D), beta0.reshape(1, D))
    e1 = _ln1_call(raw1, seg_1.astype(jnp.int32), pos_table[:S], seg_table,
                   gamma1.reshape(1, D), beta1.reshape(1, D))
    return (e0, e1)
